# baseline (speedup 1.0000x reference)
import functools

import jax
import jax.numpy as jnp
from jax import lax
from jax.experimental import pallas as pl
from jax.experimental.pallas import tpu as pltpu
from jax.experimental.pallas import tpu_sc as plsc

VOCAB = 100000
DIM = 64
BATCH = 16384

NUM_CORES = 2
NUM_SUBCORES = 16
LANES = 16
NW = NUM_CORES * NUM_SUBCORES
BPW = BATCH // NW
CHUNK = 128
NCHUNK = BPW // CHUNK


def _sqrt_vec(x):
    xi = lax.bitcast_convert_type(x, jnp.int32)
    y = lax.bitcast_convert_type(jnp.int32(0x5F3759DF) - (xi >> 1), jnp.float32)
    for _ in range(3):
        y = y * (1.5 - 0.5 * x * y * y)
    return x * y


def _unpack32(ref, row, kk):
    v = ref[row, pl.ds(kk * 2 * LANES, 2 * LANES)]
    return plsc.unpack(v, format=plsc.PackFormat.INTERLEAVED,
                       preferred_element_type=jnp.float32)


def _sc_kernel(w_idx_hbm, c_idx_hbm, w_hbm, c_hbm,
               score_hbm, sw_hbm, sc_hbm,
               widx_v, cidx_v, wrows_v, crows_v,
               s_v, sw_v, sc_v, sem):
    wid = lax.axis_index("s") * NUM_CORES + lax.axis_index("c")
    base = wid * BPW

    for k in range(NCHUNK):
        pltpu.sync_copy(w_idx_hbm.at[pl.ds(base + k * CHUNK, CHUNK)],
                        widx_v.at[k])
        pltpu.sync_copy(c_idx_hbm.at[pl.ds(base + k * CHUNK, CHUNK)],
                        cidx_v.at[k])

    copies = []
    for k in range(NCHUNK):
        copies.append(pltpu.async_copy(
            w_hbm.at[widx_v.at[k]], wrows_v.at[pl.ds(k * CHUNK, CHUNK)], sem))
        copies.append(pltpu.async_copy(
            c_hbm.at[cidx_v.at[k]], crows_v.at[pl.ds(k * CHUNK, CHUNK)], sem))
    for cp in copies:
        cp.wait()

    lane_iota = lax.iota(jnp.int32, LANES)

    def group_body(g, _):
        rs = jnp.zeros((LANES,), jnp.float32)
        rw = jnp.zeros((LANES,), jnp.float32)
        rc = jnp.zeros((LANES,), jnp.float32)
        for r in range(LANES):
            row = g * LANES + r
            acc_s = jnp.zeros((LANES,), jnp.float32)
            acc_w = jnp.zeros((LANES,), jnp.float32)
            acc_c = jnp.zeros((LANES,), jnp.float32)
            for kk in range(DIM // (2 * LANES)):
                wa, wb = _unpack32(wrows_v, row, kk)
                ca, cb = _unpack32(crows_v, row, kk)
                acc_s = acc_s + wa * ca + wb * cb
                acc_w = acc_w + wa * wa + wb * wb
                acc_c = acc_c + ca * ca + cb * cb
            m = lane_iota == r
            rs = jnp.where(m, jnp.sum(acc_s), rs)
            rw = jnp.where(m, jnp.sum(acc_w), rw)
            rc = jnp.where(m, jnp.sum(acc_c), rc)
        sl = pl.ds(g * LANES, LANES)
        s_v[sl] = rs
        sw_v[sl] = _sqrt_vec(rw)
        sc_v[sl] = _sqrt_vec(rc)
        return 0

    lax.fori_loop(0, BPW // LANES, group_body, 0)

    pltpu.sync_copy(s_v, score_hbm.at[pl.ds(base, BPW)])
    pltpu.sync_copy(sw_v, sw_hbm.at[pl.ds(base, BPW)])
    pltpu.sync_copy(sc_v, sc_hbm.at[pl.ds(base, BPW)])


_mesh = plsc.VectorSubcoreMesh(
    core_axis_name="c", subcore_axis_name="s",
    num_cores=NUM_CORES, num_subcores=NUM_SUBCORES)

_sc_call = functools.partial(
    pl.kernel,
    out_type=(
        jax.ShapeDtypeStruct((BATCH,), jnp.float32),
        jax.ShapeDtypeStruct((BATCH,), jnp.float32),
        jax.ShapeDtypeStruct((BATCH,), jnp.float32),
    ),
    mesh=_mesh,
    compiler_params=pltpu.CompilerParams(
        needs_layout_passes=False, use_tc_tiling_on_sc=False),
    scratch_types=[
        pltpu.VMEM((NCHUNK, CHUNK), jnp.int32),
        pltpu.VMEM((NCHUNK, CHUNK), jnp.int32),
        pltpu.VMEM((BPW, DIM), jnp.bfloat16),
        pltpu.VMEM((BPW, DIM), jnp.bfloat16),
        pltpu.VMEM((BPW,), jnp.float32),
        pltpu.VMEM((BPW,), jnp.float32),
        pltpu.VMEM((BPW,), jnp.float32),
        pltpu.SemaphoreType.DMA,
    ],
)(_sc_kernel)


@jax.jit
def kernel(w_idx, c_idx, W, C):
    w_idx = w_idx.astype(jnp.int32)
    c_idx = c_idx.astype(jnp.int32)
    Wb = W.astype(jnp.bfloat16)
    Cb = C.astype(jnp.bfloat16)
    return _sc_call(w_idx, c_idx, Wb, Cb)


# speedup vs baseline: 1.5217x; 1.5217x over previous
import functools

import jax
import jax.numpy as jnp
from jax import lax
from jax.experimental import pallas as pl
from jax.experimental.pallas import tpu as pltpu
from jax.experimental.pallas import tpu_sc as plsc

VOCAB = 100000
DIM = 64
BATCH = 16384

NUM_CORES = 2
NUM_SUBCORES = 16
LANES = 16
NW = NUM_CORES * NUM_SUBCORES
BPW = BATCH // NW
CHUNK = 128
NCHUNK = BPW // CHUNK

_mesh = plsc.VectorSubcoreMesh(
    core_axis_name="c", subcore_axis_name="s",
    num_cores=NUM_CORES, num_subcores=NUM_SUBCORES)


def _worker_base():
    wid = lax.axis_index("s") * NUM_CORES + lax.axis_index("c")
    return wid * BPW


def _sqrt_vec(x):
    xi = lax.bitcast_convert_type(x, jnp.int32)
    y = lax.bitcast_convert_type(jnp.int32(0x5F3759DF) - (xi >> 1), jnp.float32)
    for _ in range(3):
        y = y * (1.5 - 0.5 * x * y * y)
    return x * y


def _w_kernel(w_idx_hbm, w_hbm, sw_hbm, wg_hbm,
              widx_v, buf0, buf1, sw_v, sem):
    base = _worker_base()
    for k in range(NCHUNK):
        pltpu.sync_copy(w_idx_hbm.at[pl.ds(base + k * CHUNK, CHUNK)],
                        widx_v.at[k])

    lane_iota = lax.iota(jnp.int32, LANES)
    bufs = (buf0, buf1)

    def fire_chunk(k, buf):
        copies = []
        for l in range(CHUNK // LANES):
            vec = widx_v[k, pl.ds(l * LANES, LANES)]
            for r in range(LANES):
                i = jnp.sum(jnp.where(lane_iota == r, vec, 0))
                slot = l * LANES + r
                copies.append(pltpu.async_copy(
                    w_hbm.at[pl.ds(i, 1), :],
                    buf.at[pl.ds(slot, 1), :], sem))
        return copies

    pending = fire_chunk(0, bufs[0])
    for k in range(NCHUNK):
        cur = bufs[k % 2]
        nxt_pending = (fire_chunk(k + 1, bufs[(k + 1) % 2])
                       if k + 1 < NCHUNK else [])
        for cp in pending:
            cp.wait()
        pending = nxt_pending

        def group_body(g, _):
            rw = jnp.zeros((LANES,), jnp.float32)
            for r in range(LANES):
                row = g * LANES + r
                acc_w = jnp.zeros((LANES,), jnp.float32)
                for kk in range(DIM // LANES):
                    wv = cur[row, pl.ds(kk * LANES, LANES)]
                    acc_w = acc_w + wv * wv
                rw = jnp.where(lane_iota == r, jnp.sum(acc_w), rw)
            sw_v[pl.ds(k * CHUNK + g * LANES, LANES)] = _sqrt_vec(rw)
            return 0

        lax.fori_loop(0, CHUNK // LANES, group_body, 0)
        pltpu.sync_copy(cur, wg_hbm.at[pl.ds(base + k * CHUNK, CHUNK)])

    pltpu.sync_copy(sw_v, sw_hbm.at[pl.ds(base, BPW)])


def _c_kernel(c_idx_hbm, c_hbm, wg_hbm, score_hbm, sc_hbm,
              cidx_v, crows_v, wrows_v, s_v, sc_v, sem, wsem):
    base = _worker_base()
    for k in range(NCHUNK):
        pltpu.sync_copy(c_idx_hbm.at[pl.ds(base + k * CHUNK, CHUNK)],
                        cidx_v.at[k])

    copies = [pltpu.async_copy(wg_hbm.at[pl.ds(base, BPW)], wrows_v, wsem)]
    for k in range(NCHUNK):
        copies.append(pltpu.async_copy(
            c_hbm.at[cidx_v.at[k]], crows_v.at[pl.ds(k * CHUNK, CHUNK)], sem))
    for cp in copies:
        cp.wait()

    lane_iota = lax.iota(jnp.int32, LANES)

    def group_body(g, _):
        rs = jnp.zeros((LANES,), jnp.float32)
        rc = jnp.zeros((LANES,), jnp.float32)
        for r in range(LANES):
            row = g * LANES + r
            acc_s = jnp.zeros((LANES,), jnp.float32)
            acc_c = jnp.zeros((LANES,), jnp.float32)
            for kk in range(DIM // LANES):
                wv = wrows_v[row, pl.ds(kk * LANES, LANES)]
                cv = crows_v[row, pl.ds(kk * LANES, LANES)]
                acc_s = acc_s + wv * cv
                acc_c = acc_c + cv * cv
            m = lane_iota == r
            rs = jnp.where(m, jnp.sum(acc_s), rs)
            rc = jnp.where(m, jnp.sum(acc_c), rc)
        sl = pl.ds(g * LANES, LANES)
        s_v[sl] = rs
        sc_v[sl] = _sqrt_vec(rc)
        return 0

    lax.fori_loop(0, BPW // LANES, group_body, 0)

    pltpu.sync_copy(s_v, score_hbm.at[pl.ds(base, BPW)])
    pltpu.sync_copy(sc_v, sc_hbm.at[pl.ds(base, BPW)])


_w_call = functools.partial(
    pl.kernel,
    out_type=(
        jax.ShapeDtypeStruct((BATCH,), jnp.float32),
        jax.ShapeDtypeStruct((BATCH, DIM), jnp.float32),
    ),
    mesh=_mesh,
    compiler_params=pltpu.CompilerParams(
        needs_layout_passes=False, use_tc_tiling_on_sc=True),
    scratch_types=[
        pltpu.VMEM((NCHUNK, CHUNK), jnp.int32),
        pltpu.VMEM((CHUNK, DIM), jnp.float32),
        pltpu.VMEM((CHUNK, DIM), jnp.float32),
        pltpu.VMEM((BPW,), jnp.float32),
        pltpu.SemaphoreType.DMA,
    ],
)(_w_kernel)

_c_call = functools.partial(
    pl.kernel,
    out_type=(
        jax.ShapeDtypeStruct((BATCH,), jnp.float32),
        jax.ShapeDtypeStruct((BATCH,), jnp.float32),
    ),
    mesh=_mesh,
    compiler_params=pltpu.CompilerParams(
        needs_layout_passes=False, use_tc_tiling_on_sc=False),
    scratch_types=[
        pltpu.VMEM((NCHUNK, CHUNK), jnp.int32),
        pltpu.VMEM((BPW, DIM), jnp.float32),
        pltpu.VMEM((BPW, DIM), jnp.float32),
        pltpu.VMEM((BPW,), jnp.float32),
        pltpu.VMEM((BPW,), jnp.float32),
        pltpu.SemaphoreType.DMA,
        pltpu.SemaphoreType.DMA,
    ],
)(_c_kernel)


@jax.jit
def kernel(w_idx, c_idx, W, C):
    w_idx = w_idx.astype(jnp.int32)
    c_idx = c_idx.astype(jnp.int32)
    score_w, wg = _w_call(w_idx, W)
    score, score_c = _c_call(c_idx, C, wg)
    return (score, score_w, score_c)


# speedup vs baseline: 1.7064x; 1.1214x over previous
import functools

import jax
import jax.numpy as jnp
from jax import lax
from jax.experimental import pallas as pl
from jax.experimental.pallas import tpu as pltpu
from jax.experimental.pallas import tpu_sc as plsc

VOCAB = 100000
DIM = 64
BATCH = 16384

NUM_CORES = 2
NUM_SUBCORES = 16
LANES = 16
NW = NUM_CORES * NUM_SUBCORES
BPW = BATCH // NW
CHUNK = 128
NCHUNK = BPW // CHUNK

_COMPILER_PARAMS = pltpu.CompilerParams(
    needs_layout_passes=False, use_tc_tiling_on_sc=True)

_mesh = plsc.VectorSubcoreMesh(
    core_axis_name="c", subcore_axis_name="s",
    num_cores=NUM_CORES, num_subcores=NUM_SUBCORES)


def _worker_base():
    wid = lax.axis_index("s") * NUM_CORES + lax.axis_index("c")
    return wid * BPW


def _sqrt_vec(x):
    xi = lax.bitcast_convert_type(x, jnp.int32)
    y = lax.bitcast_convert_type(jnp.int32(0x5F3759DF) - (xi >> 1), jnp.float32)
    for _ in range(3):
        y = y * (1.5 - 0.5 * x * y * y)
    return x * y


_LANE_IOTA = None


def _fire_chunk(idx_v, k, table_hbm, buf, sem):
    lane_iota = lax.iota(jnp.int32, LANES)
    copies = []
    for l in range(CHUNK // LANES):
        vec = idx_v[k, pl.ds(l * LANES, LANES)]
        for r in range(LANES):
            i = jnp.sum(jnp.where(lane_iota == r, vec, 0))
            slot = l * LANES + r
            copies.append(pltpu.async_copy(
                table_hbm.at[pl.ds(i, 1), :],
                buf.at[pl.ds(slot, 1), :], sem))
    return copies


def _w_kernel(w_idx_hbm, w_hbm, sw_hbm, wg_hbm,
              widx_v, buf0, buf1, buf2, buf3, sw_v,
              sem0, sem1, sem2, sem3):
    base = _worker_base()
    for k in range(NCHUNK):
        pltpu.sync_copy(w_idx_hbm.at[pl.ds(base + k * CHUNK, CHUNK)],
                        widx_v.at[k])

    lane_iota = lax.iota(jnp.int32, LANES)
    bufs = (buf0, buf1, buf2, buf3)
    sems = (sem0, sem1, sem2, sem3)
    chunk_copies = [_fire_chunk(widx_v, k, w_hbm, bufs[k], sems[k])
                    for k in range(NCHUNK)]
    for k in range(NCHUNK):
        cur = bufs[k]
        for cp in chunk_copies[k]:
            cp.wait()

        def group_body(g, _):
            rw = jnp.zeros((LANES,), jnp.float32)
            for r in range(LANES):
                row = g * LANES + r
                acc_w = jnp.zeros((LANES,), jnp.float32)
                for kk in range(DIM // LANES):
                    wv = cur[row, pl.ds(kk * LANES, LANES)]
                    acc_w = acc_w + wv * wv
                rw = jnp.where(lane_iota == r, jnp.sum(acc_w), rw)
            sw_v[pl.ds(k * CHUNK + g * LANES, LANES)] = _sqrt_vec(rw)
            return 0

        lax.fori_loop(0, CHUNK // LANES, group_body, 0)
        pltpu.sync_copy(cur, wg_hbm.at[pl.ds(base + k * CHUNK, CHUNK)])

    pltpu.sync_copy(sw_v, sw_hbm.at[pl.ds(base, BPW)])


def _c_kernel(c_idx_hbm, c_hbm, wg_hbm, score_hbm, sc_hbm,
              cidx_v, buf0, buf1, buf2, buf3, wbuf0, wbuf1, s_v, sc_v,
              sem0, sem1, sem2, sem3, wsem):
    base = _worker_base()
    for k in range(NCHUNK):
        pltpu.sync_copy(c_idx_hbm.at[pl.ds(base + k * CHUNK, CHUNK)],
                        cidx_v.at[k])

    lane_iota = lax.iota(jnp.int32, LANES)
    bufs = (buf0, buf1, buf2, buf3)
    sems = (sem0, sem1, sem2, sem3)
    wbufs = (wbuf0, wbuf1)

    def fire_w(k, dst):
        return pltpu.async_copy(
            wg_hbm.at[pl.ds(base + k * CHUNK, CHUNK)], dst, wsem)

    chunk_copies = [_fire_chunk(cidx_v, k, c_hbm, bufs[k], sems[k])
                    for k in range(NCHUNK)]
    wpending = fire_w(0, wbufs[0])
    for k in range(NCHUNK):
        cur = bufs[k]
        wcur = wbufs[k % 2]
        nxt_wpending = (fire_w(k + 1, wbufs[(k + 1) % 2])
                        if k + 1 < NCHUNK else None)
        for cp in chunk_copies[k]:
            cp.wait()
        wpending.wait()
        wpending = nxt_wpending

        def group_body(g, _):
            rs = jnp.zeros((LANES,), jnp.float32)
            rc = jnp.zeros((LANES,), jnp.float32)
            for r in range(LANES):
                row = g * LANES + r
                acc_s = jnp.zeros((LANES,), jnp.float32)
                acc_c = jnp.zeros((LANES,), jnp.float32)
                for kk in range(DIM // LANES):
                    wv = wcur[row, pl.ds(kk * LANES, LANES)]
                    cv = cur[row, pl.ds(kk * LANES, LANES)]
                    acc_s = acc_s + wv * cv
                    acc_c = acc_c + cv * cv
                m = lane_iota == r
                rs = jnp.where(m, jnp.sum(acc_s), rs)
                rc = jnp.where(m, jnp.sum(acc_c), rc)
            sl = pl.ds(k * CHUNK + g * LANES, LANES)
            s_v[sl] = rs
            sc_v[sl] = _sqrt_vec(rc)
            return 0

        lax.fori_loop(0, CHUNK // LANES, group_body, 0)

    pltpu.sync_copy(s_v, score_hbm.at[pl.ds(base, BPW)])
    pltpu.sync_copy(sc_v, sc_hbm.at[pl.ds(base, BPW)])


_w_call = functools.partial(
    pl.kernel,
    out_type=(
        jax.ShapeDtypeStruct((BATCH,), jnp.float32),
        jax.ShapeDtypeStruct((BATCH, DIM), jnp.float32),
    ),
    mesh=_mesh,
    compiler_params=_COMPILER_PARAMS,
    scratch_types=[
        pltpu.VMEM((NCHUNK, CHUNK), jnp.int32),
        pltpu.VMEM((CHUNK, DIM), jnp.float32),
        pltpu.VMEM((CHUNK, DIM), jnp.float32),
        pltpu.VMEM((CHUNK, DIM), jnp.float32),
        pltpu.VMEM((CHUNK, DIM), jnp.float32),
        pltpu.VMEM((BPW,), jnp.float32),
        pltpu.SemaphoreType.DMA,
        pltpu.SemaphoreType.DMA,
        pltpu.SemaphoreType.DMA,
        pltpu.SemaphoreType.DMA,
    ],
)(_w_kernel)

_c_call = functools.partial(
    pl.kernel,
    out_type=(
        jax.ShapeDtypeStruct((BATCH,), jnp.float32),
        jax.ShapeDtypeStruct((BATCH,), jnp.float32),
    ),
    mesh=_mesh,
    compiler_params=_COMPILER_PARAMS,
    scratch_types=[
        pltpu.VMEM((NCHUNK, CHUNK), jnp.int32),
        pltpu.VMEM((CHUNK, DIM), jnp.float32),
        pltpu.VMEM((CHUNK, DIM), jnp.float32),
        pltpu.VMEM((CHUNK, DIM), jnp.float32),
        pltpu.VMEM((CHUNK, DIM), jnp.float32),
        pltpu.VMEM((CHUNK, DIM), jnp.float32),
        pltpu.VMEM((CHUNK, DIM), jnp.float32),
        pltpu.VMEM((BPW,), jnp.float32),
        pltpu.VMEM((BPW,), jnp.float32),
        pltpu.SemaphoreType.DMA,
        pltpu.SemaphoreType.DMA,
        pltpu.SemaphoreType.DMA,
        pltpu.SemaphoreType.DMA,
        pltpu.SemaphoreType.DMA,
    ],
)(_c_kernel)


@jax.jit
def kernel(w_idx, c_idx, W, C):
    w_idx = w_idx.astype(jnp.int32)
    c_idx = c_idx.astype(jnp.int32)
    score_w, wg = _w_call(w_idx, W)
    score, score_c = _c_call(c_idx, C, wg)
    return (score, score_w, score_c)


# speedup vs baseline: 1.7105x; 1.0024x over previous
import functools

import jax
import jax.numpy as jnp
from jax import lax
from jax.experimental import pallas as pl
from jax.experimental.pallas import tpu as pltpu
from jax.experimental.pallas import tpu_sc as plsc

VOCAB = 100000
DIM = 64
BATCH = 16384

NUM_CORES = 2
NUM_SUBCORES = 16
LANES = 16
NW = NUM_CORES * NUM_SUBCORES
BPW = BATCH // NW
CHUNK = 128
NCHUNK = BPW // CHUNK

_COMPILER_PARAMS = pltpu.CompilerParams(
    needs_layout_passes=False, use_tc_tiling_on_sc=True)

_mesh = plsc.VectorSubcoreMesh(
    core_axis_name="c", subcore_axis_name="s",
    num_cores=NUM_CORES, num_subcores=NUM_SUBCORES)


def _worker_base():
    wid = lax.axis_index("s") * NUM_CORES + lax.axis_index("c")
    return wid * BPW


def _sqrt_vec(x):
    xi = lax.bitcast_convert_type(x, jnp.int32)
    y = lax.bitcast_convert_type(jnp.int32(0x5F3759DF) - (xi >> 1), jnp.float32)
    for _ in range(3):
        y = y * (1.5 - 0.5 * x * y * y)
    return x * y


def _fire_chunk(idx_v, k, table_hbm, buf, sem):
    lane_iota = lax.iota(jnp.int32, LANES)
    copies = []
    for l in range(CHUNK // LANES):
        vec = idx_v[k, pl.ds(l * LANES, LANES)]
        for r in range(LANES):
            i = jnp.sum(jnp.where(lane_iota == r, vec, 0))
            slot = l * LANES + r
            copies.append(pltpu.async_copy(
                table_hbm.at[pl.ds(i, 1), :],
                buf.at[pl.ds(slot, 1), :], sem))
    return copies


def _w_kernel(w_idx_hbm, w_hbm, sw_hbm, wg_hbm,
              widx_v, buf0, buf1, buf2, buf3, sw_v,
              sem0, sem1, sem2, sem3):
    base = _worker_base()
    for k in range(NCHUNK):
        pltpu.sync_copy(w_idx_hbm.at[pl.ds(base + k * CHUNK, CHUNK)],
                        widx_v.at[k])

    lane_iota = lax.iota(jnp.int32, LANES)
    bufs = (buf0, buf1, buf2, buf3)
    sems = (sem0, sem1, sem2, sem3)
    chunk_copies = [_fire_chunk(widx_v, k, w_hbm, bufs[k], sems[k])
                    for k in range(NCHUNK)]
    for k in range(NCHUNK):
        cur = bufs[k]
        for cp in chunk_copies[k]:
            cp.wait()

        def group_body(g, _):
            rw = jnp.zeros((LANES,), jnp.float32)
            for r in range(LANES):
                row = g * LANES + r
                acc_w = jnp.zeros((LANES,), jnp.float32)
                for kk in range(DIM // LANES):
                    wv = cur[row, pl.ds(kk * LANES, LANES)]
                    acc_w = acc_w + wv * wv
                rw = jnp.where(lane_iota == r, jnp.sum(acc_w), rw)
            sw_v[pl.ds(k * CHUNK + g * LANES, LANES)] = _sqrt_vec(rw)
            return 0

        lax.fori_loop(0, CHUNK // LANES, group_body, 0)
        pltpu.sync_copy(cur, wg_hbm.at[pl.ds(base + k * CHUNK, CHUNK)])

    pltpu.sync_copy(sw_v, sw_hbm.at[pl.ds(base, BPW)])


def _c_kernel(c_idx_hbm, c_hbm, wg_hbm, score_hbm, sc_hbm,
              cidx_v, buf0, buf1, buf2, buf3, wbuf0, wbuf1, s_v, sc_v,
              sem0, sem1, sem2, sem3, wsem):
    base = _worker_base()
    for k in range(NCHUNK):
        pltpu.sync_copy(c_idx_hbm.at[pl.ds(base + k * CHUNK, CHUNK)],
                        cidx_v.at[k])

    lane_iota = lax.iota(jnp.int32, LANES)
    bufs = (buf0, buf1, buf2, buf3)
    sems = (sem0, sem1, sem2, sem3)
    wbufs = (wbuf0, wbuf1)

    def fire_w(k, dst):
        return pltpu.async_copy(
            wg_hbm.at[pl.ds(base + k * CHUNK, CHUNK)], dst, wsem)

    chunk_copies = [_fire_chunk(cidx_v, k, c_hbm, bufs[k], sems[k])
                    for k in range(NCHUNK)]
    wpending = fire_w(0, wbufs[0])
    for k in range(NCHUNK):
        cur = bufs[k]
        wcur = wbufs[k % 2]
        nxt_wpending = (fire_w(k + 1, wbufs[(k + 1) % 2])
                        if k + 1 < NCHUNK else None)
        for cp in chunk_copies[k]:
            cp.wait()
        wpending.wait()
        wpending = nxt_wpending

        def group_body(g, _):
            rs = jnp.zeros((LANES,), jnp.float32)
            rc = jnp.zeros((LANES,), jnp.float32)
            for r in range(LANES):
                row = g * LANES + r
                acc_s = jnp.zeros((LANES,), jnp.float32)
                acc_c = jnp.zeros((LANES,), jnp.float32)
                for kk in range(DIM // LANES):
                    wv = wcur[row, pl.ds(kk * LANES, LANES)]
                    cv = cur[row, pl.ds(kk * LANES, LANES)]
                    acc_s = acc_s + wv * cv
                    acc_c = acc_c + cv * cv
                m = lane_iota == r
                rs = jnp.where(m, jnp.sum(acc_s), rs)
                rc = jnp.where(m, jnp.sum(acc_c), rc)
            sl = pl.ds(k * CHUNK + g * LANES, LANES)
            s_v[sl] = rs
            sc_v[sl] = _sqrt_vec(rc)
            return 0

        lax.fori_loop(0, CHUNK // LANES, group_body, 0)

    pltpu.sync_copy(s_v, score_hbm.at[pl.ds(base, BPW)])
    pltpu.sync_copy(sc_v, sc_hbm.at[pl.ds(base, BPW)])


_w_call = functools.partial(
    pl.kernel,
    out_type=(
        jax.ShapeDtypeStruct((BATCH,), jnp.float32),
        jax.ShapeDtypeStruct((BATCH, DIM), jnp.float32),
    ),
    mesh=_mesh,
    compiler_params=_COMPILER_PARAMS,
    scratch_types=[
        pltpu.VMEM((NCHUNK, CHUNK), jnp.int32),
        pltpu.VMEM((CHUNK, DIM), jnp.float32),
        pltpu.VMEM((CHUNK, DIM), jnp.float32),
        pltpu.VMEM((CHUNK, DIM), jnp.float32),
        pltpu.VMEM((CHUNK, DIM), jnp.float32),
        pltpu.VMEM((BPW,), jnp.float32),
        pltpu.SemaphoreType.DMA,
        pltpu.SemaphoreType.DMA,
        pltpu.SemaphoreType.DMA,
        pltpu.SemaphoreType.DMA,
    ],
)(_w_kernel)

_c_call = functools.partial(
    pl.kernel,
    out_type=(
        jax.ShapeDtypeStruct((BATCH,), jnp.float32),
        jax.ShapeDtypeStruct((BATCH,), jnp.float32),
    ),
    mesh=_mesh,
    compiler_params=_COMPILER_PARAMS,
    scratch_types=[
        pltpu.VMEM((NCHUNK, CHUNK), jnp.int32),
        pltpu.VMEM((CHUNK, DIM), jnp.float32),
        pltpu.VMEM((CHUNK, DIM), jnp.float32),
        pltpu.VMEM((CHUNK, DIM), jnp.float32),
        pltpu.VMEM((CHUNK, DIM), jnp.float32),
        pltpu.VMEM((CHUNK, DIM), jnp.float32),
        pltpu.VMEM((CHUNK, DIM), jnp.float32),
        pltpu.VMEM((BPW,), jnp.float32),
        pltpu.VMEM((BPW,), jnp.float32),
        pltpu.SemaphoreType.DMA,
        pltpu.SemaphoreType.DMA,
        pltpu.SemaphoreType.DMA,
        pltpu.SemaphoreType.DMA,
        pltpu.SemaphoreType.DMA,
    ],
)(_c_kernel)


@jax.jit
def kernel(w_idx, c_idx, W, C):
    w_idx = w_idx.astype(jnp.int32)
    c_idx = c_idx.astype(jnp.int32)
    score_w, wg = _w_call(w_idx, W)
    score, score_c = _c_call(c_idx, C, wg)
    return (score, score_w, score_c)
